# Initial kernel scaffold; baseline (speedup 1.0000x reference)
import jax
import jax.numpy as jnp
from jax.experimental import pallas as pl


def kernel(x):
    raise NotImplementedError("write your pallas kernel here")


# speedup vs baseline: 41.5420x; 41.5420x over previous
import functools
import jax
import jax.numpy as jnp
from jax import lax
from jax.experimental import pallas as pl
from jax.experimental.pallas import tpu as pltpu

N = 256
K = 200.0


def _sample_kernel(x_ref, o_ref):
    X = x_ref[0]
    f32 = jnp.float32
    m = (X > 0.5).astype(f32)
    w = X * m

    ones = jnp.ones((1, N), f32)
    hp = jax.lax.Precision.HIGHEST
    col_cnt = lax.dot_general(ones, m, (((1,), (0,)), ((), ())))
    col_wsum = lax.dot_general(ones, w, (((1,), (0,)), ((), ())), precision=hp)
    row_cnt = lax.dot_general(ones, m, (((1,), (1,)), ((), ())))
    row_wsum = lax.dot_general(ones, w, (((1,), (1,)), ((), ())), precision=hp)
    count = jnp.sum(col_cnt)

    iu = lax.broadcasted_iota(jnp.int32, (N, N), 0)
    ju = lax.broadcasted_iota(jnp.int32, (N, N), 1)
    U = (iu <= ju).astype(f32)
    jv = lax.broadcasted_iota(jnp.int32, (1, N), 1).astype(f32)

    def direction(cnt, wsum, line_contract_dim, largest):
        prefix = lax.dot_general(cnt, U, (((1,), (0,)), ((), ())))
        total = count
        if largest:
            suf = total - prefix + cnt
            sel = suf >= K
            c = jnp.max(jnp.where(sel, jv, -1.0))
            onehot = (jv == c).astype(f32)
            pc = jnp.sum(prefix * onehot)
            n_outside = total - pc
            full_m = (jv > c).astype(f32)
        else:
            sel = prefix >= K
            c = jnp.min(jnp.where(sel, jv, 256.0))
            onehot = (jv == c).astype(f32)
            pc = jnp.sum(prefix * onehot)
            cc = jnp.sum(cnt * onehot)
            n_outside = pc - cc
            full_m = (jv < c).astype(f32)
        r = K - n_outside
        num_full = jnp.sum(wsum * jv * full_m)
        den_full = jnp.sum(wsum * full_m)
        line = lax.dot_general(onehot, X, (((1,), (line_contract_dim,)), ((), ())),
                               precision=hp)
        lm = (line > 0.5).astype(f32)
        rank = lax.dot_general(lm, U, (((1,), (0,)), ((), ())))
        takesel = lm * (rank <= r).astype(f32)
        pw = jnp.sum(line * takesel)
        return (num_full + c * pw) / (den_full + pw)

    x_max = direction(col_cnt, col_wsum, 1, True)
    x_min = direction(col_cnt, col_wsum, 1, False)
    y_max = direction(row_cnt, row_wsum, 0, True)
    y_min = direction(row_cnt, row_wsum, 0, False)

    y_min, y_max = 255.0 - y_max, 255.0 - y_min
    z = 1.0 + y_min / 128.0
    x_min = x_min - 128.0
    x_max = x_max - 128.0
    f = 221.0
    x3min = x_min / f / z
    x3max = x_max / f / z
    y3min = y_min / f / z
    y3max = y_max / f / z
    x_size = (x3max - x3min) / 2.0
    y_size = (y3max - y3min) / 2.0
    x_center = (x3max + x3min) / 2.0
    y_center = (y3max + y3min) / 2.0

    lane = lax.broadcasted_iota(jnp.int32, (1, 128), 1)
    out = (x_center * (lane == 0).astype(f32) + y_center * (lane == 1).astype(f32)
           + z * (lane == 2).astype(f32) + x_size * (lane == 3).astype(f32)
           + y_size * (lane == 4).astype(f32) + 0.1 * (lane == 5).astype(f32))
    out = jnp.where(count > 400.0, out, 0.0)
    o_ref[...] = out[None]


@jax.jit
def kernel(x):
    B = x.shape[0]
    out = pl.pallas_call(
        _sample_kernel,
        grid=(B,),
        in_specs=[pl.BlockSpec((1, N, N), lambda b: (b, 0, 0))],
        out_specs=pl.BlockSpec((1, 1, 128), lambda b: (b, 0, 0)),
        out_shape=jax.ShapeDtypeStruct((B, 1, 128), jnp.float32),
        compiler_params=pltpu.CompilerParams(
            dimension_semantics=("arbitrary",),
        ),
    )(x)
    return out.reshape(B, 128)[:, :7]


# speedup vs baseline: 168.4671x; 4.0553x over previous
import functools
import jax
import jax.numpy as jnp
from jax import lax
from jax.experimental import pallas as pl
from jax.experimental.pallas import tpu as pltpu
from jax.experimental.pallas import tpu_sc as plsc

N = 256
B = 64
K = 200.0

SPB = 8


def _hist_kernel(x_ref, ccnt_ref, cwsum_ref, rcnt_ref, rwsum_ref,
                 rc_scr, rw_scr):
    f32 = jnp.float32
    for s in range(SPB):
        X = x_ref[s]
        m = (X > 0.5).astype(f32)
        w = X * m
        ccnt_ref[pl.ds(s, 1), :] = jnp.sum(m, axis=0, keepdims=True)
        cwsum_ref[pl.ds(s, 1), :] = jnp.sum(w, axis=0, keepdims=True)
        rc_scr[:, pl.ds(s, 1)] = jnp.sum(m, axis=1, keepdims=True)
        rw_scr[:, pl.ds(s, 1)] = jnp.sum(w, axis=1, keepdims=True)
    iu = lax.broadcasted_iota(jnp.int32, (N, N), 0)
    ju = lax.broadcasted_iota(jnp.int32, (N, N), 1)
    ident = (iu == ju).astype(f32)
    hp = jax.lax.Precision.HIGHEST
    rcnt_ref[...] = lax.dot_general(rc_scr[...], ident, (((0,), (0,)), ((), ())))
    rwsum_ref[...] = lax.dot_general(rw_scr[...], ident, (((0,), (0,)), ((), ())),
                                     precision=hp)


def _histograms(x):
    out = pl.pallas_call(
        _hist_kernel,
        grid=(B // SPB,),
        in_specs=[pl.BlockSpec((SPB, N, N), lambda g: (g, 0, 0))],
        out_specs=[pl.BlockSpec((SPB, N), lambda g: (g, 0))] * 4,
        out_shape=[jax.ShapeDtypeStruct((B, N), jnp.float32)] * 4,
        scratch_shapes=[pltpu.VMEM((N, SPB), jnp.float32)] * 2,
        compiler_params=pltpu.CompilerParams(
            dimension_semantics=("arbitrary",),
        ),
    )(x)
    return out


_NC, _NS, _L = 2, 16, 16
_NW = _NC * _NS
_SPW = B // _NW


def _sc_body(xf_hbm, hist_hbm, out_hbm,
             hist_v, pfx_v, lines_v,
             idx0_v, idx1_v, idx2_v, idx3_v, out_v, sem):
    f32 = jnp.float32
    wid = lax.axis_index("s") * _NC + lax.axis_index("c")
    lane = lax.iota(jnp.int32, _L)
    lanef = lane.astype(f32)

    for s in range(_SPW):
        b = wid * _SPW + s

        pltpu.sync_copy(hist_hbm.at[b], hist_v)

        def build_prefix(cnt_off, pfx_off):
            run = jnp.float32(0.0)
            for k in range(16):
                ch = hist_v[pl.ds(cnt_off + 16 * k, 16)]
                pfx_v[pl.ds(pfx_off + 16 * k, 16)] = plsc.cumsum(ch) + run
                run = run + jnp.sum(ch)
            return run

        def find_cut(cnt_off, pfx_off, total, largest):
            c = jnp.float32(-1.0) if largest else jnp.float32(256.0)
            for k in range(16):
                p = pfx_v[pl.ds(pfx_off + 16 * k, 16)]
                cn = hist_v[pl.ds(cnt_off + 16 * k, 16)]
                jg = lanef + jnp.float32(16 * k)
                if largest:
                    cand = jnp.where((total - p + cn) >= K, jg, -1.0)
                    c = jnp.maximum(c, jnp.max(cand))
                else:
                    cand = jnp.where(p >= K, jg, 256.0)
                    c = jnp.minimum(c, jnp.min(cand))
            return jnp.clip(c, 0.0, 255.0)

        def residual_and_full(cnt_off, wsum_off, pfx_off, total, c, largest):
            pc = jnp.float32(0.0)
            cc = jnp.float32(0.0)
            numv = jnp.zeros((_L,), f32)
            denv = jnp.zeros((_L,), f32)
            for k in range(16):
                jg = lanef + jnp.float32(16 * k)
                oh = (jg == c).astype(f32)
                p = pfx_v[pl.ds(pfx_off + 16 * k, 16)]
                cn = hist_v[pl.ds(cnt_off + 16 * k, 16)]
                wc = hist_v[pl.ds(wsum_off + 16 * k, 16)]
                pc = pc + jnp.sum(p * oh)
                cc = cc + jnp.sum(cn * oh)
                fm = (jg > c).astype(f32) if largest else (jg < c).astype(f32)
                denv = denv + wc * fm
                numv = numv + wc * jg * fm
            n_out = (total - pc) if largest else (pc - cc)
            r = K - n_out
            return r, jnp.sum(numv), jnp.sum(denv)

        total = build_prefix(0, 0)
        c_xmax = find_cut(0, 0, total, True)
        c_xmin = find_cut(0, 0, total, False)

        sample_base = b * (N * N)
        copies = []
        for d, (cf, idx_lo, idx_hi) in enumerate(
                [(c_xmax, idx0_v, idx1_v), (c_xmin, idx2_v, idx3_v)]):
            ci = cf.astype(jnp.int32)
            base = sample_base + ci
            for k in range(8):
                idx_lo[pl.ds(16 * k, 16)] = base + N * (lane + 16 * k)
            for k in range(8):
                idx_hi[pl.ds(16 * k, 16)] = base + N * (lane + 16 * k + 128)
            off = d * N
            copies.append(pltpu.async_copy(
                xf_hbm.at[idx_lo], lines_v.at[pl.ds(off, 128)], sem))
            copies.append(pltpu.async_copy(
                xf_hbm.at[idx_hi], lines_v.at[pl.ds(off + 128, 128)], sem))

        build_prefix(512, 256)
        c_ymax = find_cut(512, 256, total, True)
        c_ymin = find_cut(512, 256, total, False)
        for d, cf in enumerate([c_ymax, c_ymin]):
            ci = cf.astype(jnp.int32)
            start = sample_base + ci * N
            copies.append(pltpu.async_copy(
                xf_hbm.at[pl.ds(start, N)], lines_v.at[pl.ds((2 + d) * N, N)],
                sem))

        r_xmax, num_xmax, den_xmax = residual_and_full(0, 256, 0, total, c_xmax, True)
        r_xmin, num_xmin, den_xmin = residual_and_full(0, 256, 0, total, c_xmin, False)
        r_ymax, num_ymax, den_ymax = residual_and_full(512, 768, 256, total, c_ymax, True)
        r_ymin, num_ymin, den_ymin = residual_and_full(512, 768, 256, total, c_ymin, False)

        for cp in copies:
            cp.wait()

        def prefix_select(line_off, r):
            run = jnp.float32(0.0)
            accv = jnp.zeros((_L,), f32)
            for k in range(16):
                v = lines_v[pl.ds(line_off + 16 * k, 16)]
                mf = (v > 0.5).astype(f32)
                rank = plsc.cumsum(mf) + run
                take = jnp.logical_and(mf > 0.0, rank <= r)
                accv = accv + jnp.where(take, v, 0.0)
                run = jnp.max(rank)
            return jnp.sum(accv)

        pw_xmax = prefix_select(0, r_xmax)
        pw_xmin = prefix_select(N, r_xmin)
        pw_ymax = prefix_select(2 * N, r_ymax)
        pw_ymin = prefix_select(3 * N, r_ymin)

        def recip16(d):
            bits = plsc.bitcast(d, jnp.int32)
            y = plsc.bitcast(jnp.int32(0x7EF127EA) - bits, f32)
            for _ in range(4):
                y = y * (2.0 - d * y)
            return y

        def vdiv(num_s, den_s):
            zv = jnp.zeros((_L,), f32)
            return (zv + num_s) * recip16(zv + den_s)

        x_max = vdiv(num_xmax + c_xmax * pw_xmax, den_xmax + pw_xmax)
        x_min = vdiv(num_xmin + c_xmin * pw_xmin, den_xmin + pw_xmin)
        y_max = vdiv(num_ymax + c_ymax * pw_ymax, den_ymax + pw_ymax)
        y_min = vdiv(num_ymin + c_ymin * pw_ymin, den_ymin + pw_ymin)

        y_min, y_max = 255.0 - y_max, 255.0 - y_min
        z = 1.0 + y_min * (1.0 / 128.0)
        x_min = x_min - 128.0
        x_max = x_max - 128.0
        inv = recip16(221.0 * z)
        x3min = x_min * inv
        x3max = x_max * inv
        y3min = y_min * inv
        y3max = y_max * inv
        x_size = (x3max - x3min) * 0.5
        y_size = (y3max - y3min) * 0.5
        x_center = (x3max + x3min) * 0.5
        y_center = (y3max + y3min) * 0.5

        vals = jnp.where(lane == 0, x_center,
               jnp.where(lane == 1, y_center,
               jnp.where(lane == 2, z,
               jnp.where(lane == 3, x_size,
               jnp.where(lane == 4, y_size,
               jnp.where(lane == 5, jnp.float32(0.1), jnp.float32(0.0)))))))
        totv = jnp.zeros((_L,), f32) + total
        out_v[...] = jnp.where(totv > 400.0, vals, jnp.float32(0.0))
        pltpu.sync_copy(out_v, out_hbm.at[b])


@functools.cache
def _sc_stage():
    return pl.kernel(
        _sc_body,
        out_type=jax.ShapeDtypeStruct((B, _L), jnp.float32),
        mesh=plsc.VectorSubcoreMesh(core_axis_name="c", subcore_axis_name="s"),
        compiler_params=pltpu.CompilerParams(needs_layout_passes=False),
        scratch_types=[
            pltpu.VMEM((4 * N,), jnp.float32),
            pltpu.VMEM((2 * N,), jnp.float32),
            pltpu.VMEM((4 * N,), jnp.float32),
            pltpu.VMEM((128,), jnp.int32),
            pltpu.VMEM((128,), jnp.int32),
            pltpu.VMEM((128,), jnp.int32),
            pltpu.VMEM((128,), jnp.int32),
            pltpu.VMEM((_L,), jnp.float32),
            pltpu.SemaphoreType.DMA,
        ],
    )


@jax.jit
def kernel(x):
    ccnt, cwsum, rcnt, rwsum = _histograms(x)
    hist = jnp.concatenate([ccnt, cwsum, rcnt, rwsum], axis=1)
    out = _sc_stage()(x.reshape(-1), hist)
    return out[:, :7]
